# baseline (speedup 1.0000x reference)
import jax
import jax.numpy as jnp
from jax.experimental import pallas as pl
from jax.experimental.pallas import tpu as pltpu

N_BLOCK = 256


def _cluster_kernel(embs_ref, centers_ref, ids_ref, loss_ref, c2_ref):
    c = centers_ref[...]

    @pl.when(pl.program_id(0) == 0)
    def _c2():
        c2_ref[...] = jnp.sum(c * c, axis=1)[None, :]

    e = embs_ref[...]
    dot = jax.lax.dot_general(
        e, c, (((1,), (1,)), ((), ())),
        preferred_element_type=jnp.float32)
    e2 = jnp.sum(e * e, axis=1, keepdims=True)
    d2 = e2 - 2.0 * dot + c2_ref[...]
    d2 = jnp.maximum(d2, 0.0)
    dists = jnp.sqrt(d2 + 1e-12)
    row_min = jnp.min(dists, axis=1, keepdims=True)
    k = dists.shape[1]
    idx = jax.lax.broadcasted_iota(jnp.int32, dists.shape, 1)
    ids = jnp.min(jnp.where(dists == row_min, idx, k), axis=1)
    ids_ref[...] = ids[None, None, :]
    partial = jnp.sum(row_min)

    @pl.when(pl.program_id(0) == 0)
    def _init():
        loss_ref[0, 0] = partial

    @pl.when(pl.program_id(0) != 0)
    def _acc():
        loss_ref[0, 0] += partial


def kernel(embs, centers):
    n, d = embs.shape
    k = centers.shape[0]
    grid = n // N_BLOCK
    ids, loss = pl.pallas_call(
        _cluster_kernel,
        grid=(grid,),
        in_specs=[
            pl.BlockSpec((N_BLOCK, d), lambda i: (i, 0)),
            pl.BlockSpec((k, d), lambda i: (0, 0)),
        ],
        out_specs=[
            pl.BlockSpec((1, 1, N_BLOCK), lambda i: (i, 0, 0)),
            pl.BlockSpec((1, 1), lambda i: (0, 0), memory_space=pltpu.SMEM),
        ],
        out_shape=[
            jax.ShapeDtypeStruct((grid, 1, N_BLOCK), jnp.int32),
            jax.ShapeDtypeStruct((1, 1), jnp.float32),
        ],
        scratch_shapes=[pltpu.VMEM((1, k), jnp.float32)],
    )(embs, centers)
    return (centers, ids.reshape(n), loss[0, 0])


# speedup vs baseline: 1.8350x; 1.8350x over previous
import jax
import jax.numpy as jnp
from jax.experimental import pallas as pl
from jax.experimental.pallas import tpu as pltpu

N_BLOCK = 256


def _cluster_kernel(embs_ref, centers_ref, ids_ref, loss_ref, c2_ref):
    c = centers_ref[...]

    @pl.when(pl.program_id(0) == 0)
    def _c2():
        c2_ref[...] = jnp.sum(c * c, axis=1)[None, :]

    e = embs_ref[...]
    dot = jax.lax.dot_general(
        e, c, (((1,), (1,)), ((), ())),
        preferred_element_type=jnp.float32)
    e2 = jnp.sum(e * e, axis=1, keepdims=True)
    d2 = e2 - 2.0 * dot + c2_ref[...]
    row_min = jnp.min(d2, axis=1, keepdims=True)
    k = d2.shape[1]
    idx = jax.lax.broadcasted_iota(jnp.int32, d2.shape, 1)
    ids = jnp.min(jnp.where(d2 == row_min, idx, k), axis=1)
    ids_ref[...] = ids[None, None, :]
    mind = jnp.sqrt(jnp.maximum(row_min, 0.0) + 1e-12)
    partial = jnp.sum(mind)

    @pl.when(pl.program_id(0) == 0)
    def _init():
        loss_ref[0, 0] = partial

    @pl.when(pl.program_id(0) != 0)
    def _acc():
        loss_ref[0, 0] += partial


def kernel(embs, centers):
    n, d = embs.shape
    k = centers.shape[0]
    grid = n // N_BLOCK
    ids, loss = pl.pallas_call(
        _cluster_kernel,
        grid=(grid,),
        in_specs=[
            pl.BlockSpec((N_BLOCK, d), lambda i: (i, 0)),
            pl.BlockSpec((k, d), lambda i: (0, 0)),
        ],
        out_specs=[
            pl.BlockSpec((1, 1, N_BLOCK), lambda i: (i, 0, 0)),
            pl.BlockSpec((1, 1), lambda i: (0, 0), memory_space=pltpu.SMEM),
        ],
        out_shape=[
            jax.ShapeDtypeStruct((grid, 1, N_BLOCK), jnp.int32),
            jax.ShapeDtypeStruct((1, 1), jnp.float32),
        ],
        scratch_shapes=[pltpu.VMEM((1, k), jnp.float32)],
    )(embs, centers)
    return (centers, ids.reshape(n), loss[0, 0])


# speedup vs baseline: 1.9212x; 1.0470x over previous
import jax
import jax.numpy as jnp
from jax.experimental import pallas as pl
from jax.experimental.pallas import tpu as pltpu

N_BLOCK = 256


def _cluster_kernel(embs_ref, centers_ref, idx_ref, ids_ref, loss_ref, c2_ref):
    c = centers_ref[...]

    @pl.when(pl.program_id(0) == 0)
    def _c2():
        c2_ref[...] = jnp.sum(c * c, axis=1)[None, :]

    e = embs_ref[...]
    ndot2 = jax.lax.dot_general(
        e * -2.0, c, (((1,), (1,)), ((), ())),
        preferred_element_type=jnp.float32)
    e2 = jnp.sum(e * e, axis=1, keepdims=True)
    d2 = (e2 + ndot2) + c2_ref[...]
    row_min = jnp.min(d2, axis=1, keepdims=True)
    idx = idx_ref[...]
    ids_f = jnp.min(jnp.where(d2 == row_min, idx, jnp.inf), axis=1)
    ids_ref[...] = ids_f.astype(jnp.int32)[None, None, :]
    mind = jnp.sqrt(jnp.maximum(row_min, 0.0) + 1e-12)
    partial = jnp.sum(mind)

    @pl.when(pl.program_id(0) == 0)
    def _init():
        loss_ref[0, 0] = partial

    @pl.when(pl.program_id(0) != 0)
    def _acc():
        loss_ref[0, 0] += partial


def kernel(embs, centers):
    n, d = embs.shape
    k = centers.shape[0]
    grid = n // N_BLOCK
    ids, loss = pl.pallas_call(
        _cluster_kernel,
        grid=(grid,),
        in_specs=[
            pl.BlockSpec((N_BLOCK, d), lambda i: (i, 0)),
            pl.BlockSpec((k, d), lambda i: (0, 0)),
            pl.BlockSpec((1, k), lambda i: (0, 0)),
        ],
        out_specs=[
            pl.BlockSpec((1, 1, N_BLOCK), lambda i: (i, 0, 0)),
            pl.BlockSpec((1, 1), lambda i: (0, 0), memory_space=pltpu.SMEM),
        ],
        out_shape=[
            jax.ShapeDtypeStruct((grid, 1, N_BLOCK), jnp.int32),
            jax.ShapeDtypeStruct((1, 1), jnp.float32),
        ],
        scratch_shapes=[pltpu.VMEM((1, k), jnp.float32)],
    )(embs, centers, jnp.arange(k, dtype=jnp.float32)[None, :])
    return (centers, ids.reshape(n), loss[0, 0])


# speedup vs baseline: 2.2310x; 1.1612x over previous
import jax
import jax.numpy as jnp
from jax.experimental import pallas as pl
from jax.experimental.pallas import tpu as pltpu

N_BLOCK = 512


def _cluster_kernel(embs_ref, centers_ref, idx_ref, ids_ref, loss_ref, c2_ref):
    c = centers_ref[...]

    @pl.when(pl.program_id(0) == 0)
    def _c2():
        c2_ref[...] = jnp.sum(c * c, axis=1)[None, :]

    e = embs_ref[...]
    ndot2 = jax.lax.dot_general(
        e * -2.0, c, (((1,), (1,)), ((), ())),
        preferred_element_type=jnp.float32)
    e2 = jnp.sum(e * e, axis=1, keepdims=True)
    d2 = (e2 + ndot2) + c2_ref[...]
    row_min = jnp.min(d2, axis=1, keepdims=True)
    idx = idx_ref[...]
    ids_f = jnp.min(jnp.where(d2 == row_min, idx, jnp.inf), axis=1)
    ids_ref[...] = ids_f.astype(jnp.int32)[None, None, :]
    mind = jnp.sqrt(jnp.maximum(row_min, 0.0) + 1e-12)
    partial = jnp.sum(mind)

    @pl.when(pl.program_id(0) == 0)
    def _init():
        loss_ref[0, 0] = partial

    @pl.when(pl.program_id(0) != 0)
    def _acc():
        loss_ref[0, 0] += partial


def kernel(embs, centers):
    n, d = embs.shape
    k = centers.shape[0]
    grid = n // N_BLOCK
    ids, loss = pl.pallas_call(
        _cluster_kernel,
        grid=(grid,),
        in_specs=[
            pl.BlockSpec((N_BLOCK, d), lambda i: (i, 0)),
            pl.BlockSpec((k, d), lambda i: (0, 0)),
            pl.BlockSpec((1, k), lambda i: (0, 0)),
        ],
        out_specs=[
            pl.BlockSpec((1, 1, N_BLOCK), lambda i: (i, 0, 0)),
            pl.BlockSpec((1, 1), lambda i: (0, 0), memory_space=pltpu.SMEM),
        ],
        out_shape=[
            jax.ShapeDtypeStruct((grid, 1, N_BLOCK), jnp.int32),
            jax.ShapeDtypeStruct((1, 1), jnp.float32),
        ],
        scratch_shapes=[pltpu.VMEM((1, k), jnp.float32)],
    )(embs, centers, jnp.arange(k, dtype=jnp.float32)[None, :])
    return (centers, ids.reshape(n), loss[0, 0])


# speedup vs baseline: 2.4476x; 1.0971x over previous
import jax
import jax.numpy as jnp
from jax.experimental import pallas as pl
from jax.experimental.pallas import tpu as pltpu

N_BLOCK = 1024


def _cluster_kernel(embs_ref, centers_ref, idx_ref, ids_ref, loss_ref, c2_ref):
    c = centers_ref[...]

    @pl.when(pl.program_id(0) == 0)
    def _c2():
        c2_ref[...] = jnp.sum(c * c, axis=1)[None, :]

    e = embs_ref[...]
    ndot2 = jax.lax.dot_general(
        e * -2.0, c, (((1,), (1,)), ((), ())),
        preferred_element_type=jnp.float32)
    e2 = jnp.sum(e * e, axis=1, keepdims=True)
    d2 = (e2 + ndot2) + c2_ref[...]
    row_min = jnp.min(d2, axis=1, keepdims=True)
    idx = idx_ref[...]
    ids_f = jnp.min(jnp.where(d2 == row_min, idx, jnp.inf), axis=1)
    ids_ref[...] = ids_f.astype(jnp.int32)[None, None, :]
    mind = jnp.sqrt(jnp.maximum(row_min, 0.0) + 1e-12)
    partial = jnp.sum(mind)

    @pl.when(pl.program_id(0) == 0)
    def _init():
        loss_ref[0, 0] = partial

    @pl.when(pl.program_id(0) != 0)
    def _acc():
        loss_ref[0, 0] += partial


def kernel(embs, centers):
    n, d = embs.shape
    k = centers.shape[0]
    grid = n // N_BLOCK
    ids, loss = pl.pallas_call(
        _cluster_kernel,
        grid=(grid,),
        in_specs=[
            pl.BlockSpec((N_BLOCK, d), lambda i: (i, 0)),
            pl.BlockSpec((k, d), lambda i: (0, 0)),
            pl.BlockSpec((1, k), lambda i: (0, 0)),
        ],
        out_specs=[
            pl.BlockSpec((1, 1, N_BLOCK), lambda i: (i, 0, 0)),
            pl.BlockSpec((1, 1), lambda i: (0, 0), memory_space=pltpu.SMEM),
        ],
        out_shape=[
            jax.ShapeDtypeStruct((grid, 1, N_BLOCK), jnp.int32),
            jax.ShapeDtypeStruct((1, 1), jnp.float32),
        ],
        scratch_shapes=[pltpu.VMEM((1, k), jnp.float32)],
    )(embs, centers, jnp.arange(k, dtype=jnp.float32)[None, :])
    return (centers, ids.reshape(n), loss[0, 0])
